# Initial kernel scaffold; baseline (speedup 1.0000x reference)
import jax
import jax.numpy as jnp
from jax.experimental import pallas as pl


def kernel(x, edge_index, edge_attr, W, W_e, b_e, W_pre, b_pre, W_post, b_post, W_ih, b_ih, W_hh, b_hh):
    raise NotImplementedError("write your pallas kernel here")


# speedup vs baseline: 2.7481x; 2.7481x over previous
import functools
import numpy as np
import jax
import jax.numpy as jnp
from jax.experimental import pallas as pl
from jax.experimental.pallas import tpu as pltpu

_DEG_HIST = np.array([0., 10., 25., 40., 50., 45., 35., 20., 10., 5.],
                     dtype=np.float32)
_AVG_LOG = float(np.mean(np.log(_DEG_HIST + 1.0)))

_EG = 256
_Z = np.int32(0)


def _node_tables_body(x_ref, w_ref, w1_ref, w2_ref, m_ref, a_ref, b_ref):
    xm = jnp.dot(x_ref[...], w_ref[...], preferred_element_type=jnp.float32)
    m_ref[...] = xm
    a_ref[...] = jnp.dot(xm, w1_ref[...], preferred_element_type=jnp.float32)
    b_ref[...] = jnp.dot(xm, w2_ref[...], preferred_element_type=jnp.float32)


def _edge_body(eg, dst_ref, src_ref, attr_ref, a_ref, b_ref, cw_ref,
               acc1a_ref, acc1b_ref, acc2a_ref, acc2b_ref):
    first = (pl.program_id(0) == 0) & (pl.program_id(1) == 0)

    @pl.when(first)
    def _init():
        acc1a_ref[...] = jnp.zeros_like(acc1a_ref)
        acc1b_ref[...] = jnp.zeros_like(acc1b_ref)
        acc2a_ref[...] = jnp.full_like(acc2a_ref, jnp.inf)
        acc2b_ref[...] = jnp.full_like(acc2b_ref, -jnp.inf)

    w0 = cw_ref[0:1, :]
    w1 = cw_ref[1:2, :]
    w2 = cw_ref[2:3, :]
    c3 = cw_ref[3:4, :]
    bpre = cw_ref[4:5, :]

    base = pl.multiple_of(pl.program_id(1) * eg, eg)
    att = attr_ref[pl.ds(base, eg), :]
    a0 = att[:, 0:1]
    a1 = att[:, 1:2]
    a2 = att[:, 2:3]
    ew = att[:, 3:4]
    c8 = a0 * w0 + a1 * w1 + a2 * w2 + c3
    arows = [a_ref[pl.ds(dst_ref[base + j], 1), :] for j in range(eg)]
    brows = [b_ref[pl.ds(src_ref[base + j], 1), :] for j in range(eg)]
    a8 = jnp.concatenate(arows, axis=0)
    b8 = jnp.concatenate(brows, axis=0)
    msg = ew * (a8 + b8 + c8) + bpre
    pack1 = jnp.concatenate(
        [msg, msg * msg, jnp.ones_like(msg)], axis=1)

    for j in range(eg):
        d = dst_ref[base + j]
        c1 = acc1a_ref if j % 2 == 0 else acc1b_ref
        c1[pl.ds(d, 1), :] += pack1[j:j + 1, :]
        rm = msg[j:j + 1, :]
        acc2a_ref[pl.ds(d, 1), :] = jnp.minimum(
            acc2a_ref[pl.ds(d, 1), :], rm)
        acc2b_ref[pl.ds(d, 1), :] = jnp.maximum(
            acc2b_ref[pl.ds(d, 1), :], rm)


def _post_body(m_ref, x_ref, acc1a_ref, acc1b_ref, acc2a_ref, acc2b_ref,
               wpost_ref, bpost_ref, wih_ref, bih_ref, whh_ref, bhh_ref,
               o_ref):
    d = m_ref.shape[1]
    acc1 = acc1a_ref[...] + acc1b_ref[...]

    cnt = acc1[:, 2 * d:2 * d + 1]
    cc = jnp.maximum(cnt, 1.0)
    mean = acc1[:, :d] / cc
    var = jnp.maximum(acc1[:, d:2 * d] / cc - mean * mean, 0.0)
    std = jnp.sqrt(var + 1e-5)
    has = cnt > 0.0
    mn = jnp.where(has, acc2a_ref[...], 0.0)
    mx = jnp.where(has, acc2b_ref[...], 0.0)
    agg = jnp.concatenate([mean, mn, mx, std], axis=-1)
    logd = jnp.log(cc + 1.0)
    amp = logd * (1.0 / _AVG_LOG)
    att = _AVG_LOG / logd
    full = jnp.concatenate(
        [m_ref[...], agg, agg * amp, agg * att], axis=-1)
    out = jnp.dot(full, wpost_ref[...],
                  preferred_element_type=jnp.float32) + bpost_ref[...]
    x = x_ref[...]
    gi = jnp.dot(out, wih_ref[...],
                 preferred_element_type=jnp.float32) + bih_ref[...]
    gh = jnp.dot(x, whh_ref[...],
                 preferred_element_type=jnp.float32) + bhh_ref[...]
    r = jax.nn.sigmoid(gi[:, :d] + gh[:, :d])
    z = jax.nn.sigmoid(gi[:, d:2 * d] + gh[:, d:2 * d])
    n = jnp.tanh(gi[:, 2 * d:] + r * gh[:, 2 * d:])
    o_ref[...] = (1.0 - z) * n + z * x


def _pick_block(total, target):
    for nb in range(target, 0, -1):
        if total % nb == 0 and nb % 8 == 0:
            return nb
    return total


def kernel(x, edge_index, edge_attr, W, W_e, b_e, W_pre, b_pre, W_post,
           b_post, W_ih, b_ih, W_hh, b_hh):
    n, d = x.shape
    e = edge_index.shape[1]
    x = x.astype(jnp.float32)
    dst = edge_index[1].astype(jnp.int32)
    src = edge_index[0].astype(jnp.int32)
    attr = edge_attr.astype(jnp.float32)

    w1 = W_pre[:d]
    w2 = W_pre[d:2 * d]
    w3 = W_pre[2 * d:]
    we3 = jnp.dot(W_e, w3, preferred_element_type=jnp.float32)
    c3 = jnp.dot(b_e, w3, preferred_element_type=jnp.float32)
    cw = jnp.concatenate(
        [we3, c3[None, :], b_pre[None, :], jnp.zeros((3, d), jnp.float32)],
        axis=0)

    nb = _pick_block(n, 1000)
    grid_n = n // nb

    m, a_tab, b_tab = pl.pallas_call(
        _node_tables_body,
        grid=(grid_n,),
        in_specs=[
            pl.BlockSpec((nb, d), lambda i: (i, _Z)),
            pl.BlockSpec((d, d), lambda i: (_Z, _Z)),
            pl.BlockSpec((d, d), lambda i: (_Z, _Z)),
            pl.BlockSpec((d, d), lambda i: (_Z, _Z)),
        ],
        out_specs=[
            pl.BlockSpec((nb, d), lambda i: (i, _Z)),
            pl.BlockSpec((nb, d), lambda i: (i, _Z)),
            pl.BlockSpec((nb, d), lambda i: (i, _Z)),
        ],
        out_shape=[
            jax.ShapeDtypeStruct((n, d), jnp.float32),
            jax.ShapeDtypeStruct((n, d), jnp.float32),
            jax.ShapeDtypeStruct((n, d), jnp.float32),
        ],
    )(x, W, w1, w2)

    ebk = e
    for cand in (2048, 1024, 512, 256, 128):
        if e % cand == 0:
            ebk = cand
            break
    grid_e = e // ebk
    eg = _EG
    while ebk % eg:
        eg //= 2
    full_spec = pl.BlockSpec((n, d), lambda i, j: (_Z, _Z))
    acc1a, acc1b, acc2a, acc2b = pl.pallas_call(
        functools.partial(_edge_body, eg),
        grid=(grid_e, ebk // eg),
        in_specs=[
            pl.BlockSpec((ebk,), lambda i, j: (i,), memory_space=pltpu.SMEM),
            pl.BlockSpec((ebk,), lambda i, j: (i,), memory_space=pltpu.SMEM),
            pl.BlockSpec((ebk, 4), lambda i, j: (i, _Z)),
            full_spec,
            full_spec,
            pl.BlockSpec((8, d), lambda i, j: (_Z, _Z)),
        ],
        out_specs=[
            pl.BlockSpec((n, 3 * d), lambda i, j: (_Z, _Z)),
            pl.BlockSpec((n, 3 * d), lambda i, j: (_Z, _Z)),
            pl.BlockSpec((n, d), lambda i, j: (_Z, _Z)),
            pl.BlockSpec((n, d), lambda i, j: (_Z, _Z)),
        ],
        out_shape=[
            jax.ShapeDtypeStruct((n, 3 * d), jnp.float32),
            jax.ShapeDtypeStruct((n, 3 * d), jnp.float32),
            jax.ShapeDtypeStruct((n, d), jnp.float32),
            jax.ShapeDtypeStruct((n, d), jnp.float32),
        ],
        compiler_params=pltpu.CompilerParams(
            dimension_semantics=("arbitrary", "arbitrary")),
    )(dst, src, attr, a_tab, b_tab, cw)

    full_nb = pl.BlockSpec((nb, d), lambda i: (i, _Z))
    out = pl.pallas_call(
        _post_body,
        grid=(grid_n,),
        in_specs=[
            full_nb, full_nb,
            pl.BlockSpec((nb, 3 * d), lambda i: (i, _Z)),
            pl.BlockSpec((nb, 3 * d), lambda i: (i, _Z)),
            pl.BlockSpec((nb, d), lambda i: (i, _Z)),
            pl.BlockSpec((nb, d), lambda i: (i, _Z)),
            pl.BlockSpec((13 * d, d), lambda i: (_Z, _Z)),
            pl.BlockSpec((1, d), lambda i: (_Z, _Z)),
            pl.BlockSpec((d, 3 * d), lambda i: (_Z, _Z)),
            pl.BlockSpec((1, 3 * d), lambda i: (_Z, _Z)),
            pl.BlockSpec((d, 3 * d), lambda i: (_Z, _Z)),
            pl.BlockSpec((1, 3 * d), lambda i: (_Z, _Z)),
        ],
        out_specs=pl.BlockSpec((nb, d), lambda i: (i, _Z)),
        out_shape=jax.ShapeDtypeStruct((n, d), jnp.float32),
    )(m, x, acc1a, acc1b, acc2a, acc2b,
      W_post, b_post[None, :], W_ih, b_ih[None, :], W_hh, b_hh[None, :])
    return out
